# baseline (speedup 1.0000x reference)
import functools

import jax
import jax.numpy as jnp
from jax import lax
from jax.experimental import pallas as pl
from jax.experimental.pallas import tpu as pltpu
from jax.experimental.pallas import tpu_sc as plsc

_D = 64
_SCALE = 8.0
_NC = 2
_NS = 16
_NW = _NC * _NS
_CHUNK = 128
_K = 2
_ROWS = _K * _CHUNK


def _emb_body(nitems, nq, x_hbm, tab_hbm, out_hbm,
              idx_v, buf_a, buf_b, obuf_a, obuf_b,
              gsem_a, gsem_b, ssem_a, ssem_b):
    wid = lax.axis_index("s") * _NC + lax.axis_index("c")
    k0 = wid * nitems
    pltpu.sync_copy(x_hbm.at[pl.ds(k0, nitems)], idx_v)

    iota = lax.broadcasted_iota(jnp.int32, (16,), 0)

    def gstart(n, buf, gsem):
        for j in range(_K):
            pltpu.make_async_copy(
                tab_hbm.at[idx_v.at[n, j]],
                buf.at[pl.ds(j * _CHUNK, _CHUNK)],
                gsem,
            ).start()

    def gwait(buf, gsem):
        pltpu.make_async_copy(tab_hbm.at[pl.ds(0, _ROWS)], buf, gsem).wait()

    def out_dst(n):
        k = k0 + n
        t = k // nq
        q = k % nq
        return out_hbm.at[t, :, pl.ds(q * _K, _K)]

    def sstart(n, obuf, ssem):
        pltpu.make_async_copy(obuf, out_dst(n), ssem).start()

    def swait(n, obuf, ssem):
        pltpu.make_async_copy(obuf, out_dst(n), ssem).wait()

    def transpose_scale(buf, obuf):
        def body(m, _):
            rows = iota + m * 16
            jj = m // 8
            lo = (m % 8) * 16
            for c in range(_D):
                v = plsc.load_gather(buf, [rows, jnp.full((16,), c, jnp.int32)])
                obuf[c // 8, jj, c % 8, pl.ds(lo, 16)] = v * _SCALE
            return 0

        lax.fori_loop(0, _ROWS // 16, body, 0)

    gstart(0, buf_a, gsem_a)
    gwait(buf_a, gsem_a)
    gstart(1, buf_b, gsem_b)
    transpose_scale(buf_a, obuf_a)
    sstart(0, obuf_a, ssem_a)

    def pair(p, _):
        n1 = 1 + 2 * p
        gwait(buf_b, gsem_b)
        gstart(n1 + 1, buf_a, gsem_a)
        swait(n1 - 1, obuf_a, ssem_a)
        transpose_scale(buf_b, obuf_b)
        sstart(n1, obuf_b, ssem_b)

        n2 = n1 + 1
        gwait(buf_a, gsem_a)
        gstart(n2 + 1, buf_b, gsem_b)
        swait(n1, obuf_b, ssem_b)
        transpose_scale(buf_a, obuf_a)
        sstart(n2, obuf_a, ssem_a)
        return 0

    lax.fori_loop(0, (nitems - 2) // 2, pair, 0)

    nl = nitems - 1
    gwait(buf_b, gsem_b)
    swait(nl - 2, obuf_a, ssem_a)
    transpose_scale(buf_b, obuf_b)
    sstart(nl, obuf_b, ssem_b)
    swait(nl, obuf_b, ssem_b)


def kernel(x, table):
    s0, s1 = x.shape
    b_total = s0 * s1
    assert s0 % (_K * _CHUNK) == 0 and b_total % (_NW * _ROWS) == 0
    nq = s0 // _ROWS
    nitems_total = b_total // _ROWS
    nitems = nitems_total // _NW
    assert nitems >= 4 and nitems % 2 == 0
    x4 = x.T.reshape(nitems_total, _K, _CHUNK)

    mesh = plsc.VectorSubcoreMesh(core_axis_name="c", subcore_axis_name="s")
    run = functools.partial(
        pl.kernel,
        out_type=jax.ShapeDtypeStruct((s1, _D // 8, s0 // _CHUNK, 8, _CHUNK),
                                      jnp.float32),
        mesh=mesh,
        scratch_types=[
            pltpu.VMEM((nitems, _K, _CHUNK), jnp.int32),
            pltpu.VMEM((_ROWS, _D), jnp.float32),
            pltpu.VMEM((_ROWS, _D), jnp.float32),
            pltpu.VMEM((_D // 8, _K, 8, _CHUNK), jnp.float32),
            pltpu.VMEM((_D // 8, _K, 8, _CHUNK), jnp.float32),
            pltpu.SemaphoreType.DMA,
            pltpu.SemaphoreType.DMA,
            pltpu.SemaphoreType.DMA,
            pltpu.SemaphoreType.DMA,
        ],
        compiler_params=pltpu.CompilerParams(use_tc_tiling_on_sc=False,
                                             needs_layout_passes=False),
    )(functools.partial(_emb_body, nitems, nq))
    out5 = run(x4, table)
    out = out5.transpose(2, 4, 0, 1, 3).reshape(s0, s1, _D)
    return out


# speedup vs baseline: 1.7156x; 1.7156x over previous
import functools

import jax
import jax.numpy as jnp
from jax import lax
from jax.experimental import pallas as pl
from jax.experimental.pallas import tpu as pltpu
from jax.experimental.pallas import tpu_sc as plsc

_D = 64
_SCALE = 8.0
_NC = 2
_NS = 16
_NW = _NC * _NS
_CHUNK = 128
_SKEW = 129


def _emb_body(nitems, nbc, x_hbm, tab_hbm, out_hbm,
              idx_v, buf_a, buf_b, obuf_a, obuf_b,
              gsem_a, gsem_b, ssem_a, ssem_b):
    wid = lax.axis_index("s") * _NC + lax.axis_index("c")
    k0 = wid * nitems
    pltpu.sync_copy(x_hbm.at[pl.ds(k0, nitems)], idx_v)

    iota = lax.broadcasted_iota(jnp.int32, (16,), 0)

    def gather_copy(n, buf, gsem):
        return pltpu.make_async_copy(tab_hbm.at[idx_v.at[n]], buf, gsem)

    def out_dst(n):
        k = k0 + n
        tt = k // (nbc * 8)
        bc = (k // 8) % nbc
        s = k % 8
        t = tt * 8 + s
        return out_hbm.at[t, :, bc]

    def scatter_copy(n, obuf, ssem):
        return pltpu.make_async_copy(
            obuf.at[:, :, pl.ds(0, _CHUNK)], out_dst(n), ssem)

    tr_base = iota // 8
    s2_vec = iota % 8

    def transpose_scale(buf, obuf):
        def rbody(r, _):
            l_vec = jnp.full((16,), r, jnp.int32)
            for cb in range(_D // 16):
                v = buf[r, pl.ds(cb * 16, 16)]
                plsc.store_scatter(
                    obuf, [tr_base + 2 * cb, s2_vec, l_vec], v * _SCALE)
            return 0

        lax.fori_loop(0, _CHUNK, rbody, 0)

    gather_copy(0, buf_a, gsem_a).start()
    gather_copy(0, buf_a, gsem_a).wait()
    gather_copy(1, buf_b, gsem_b).start()
    transpose_scale(buf_a, obuf_a)
    scatter_copy(0, obuf_a, ssem_a).start()

    def pair(p, _):
        n1 = 1 + 2 * p
        gather_copy(n1, buf_b, gsem_b).wait()
        gather_copy(n1 + 1, buf_a, gsem_a).start()
        scatter_copy(n1 - 1, obuf_a, ssem_a).wait()
        transpose_scale(buf_b, obuf_b)
        scatter_copy(n1, obuf_b, ssem_b).start()

        n2 = n1 + 1
        gather_copy(n2, buf_a, gsem_a).wait()
        gather_copy(n2 + 1, buf_b, gsem_b).start()
        scatter_copy(n1, obuf_b, ssem_b).wait()
        transpose_scale(buf_a, obuf_a)
        scatter_copy(n2, obuf_a, ssem_a).start()
        return 0

    lax.fori_loop(0, (nitems - 2) // 2, pair, 0)

    nl = nitems - 1
    gather_copy(nl, buf_b, gsem_b).wait()
    scatter_copy(nl - 1, obuf_a, ssem_a).wait()
    transpose_scale(buf_b, obuf_b)
    scatter_copy(nl, obuf_b, ssem_b).start()
    scatter_copy(nl, obuf_b, ssem_b).wait()


def kernel(x, table):
    s0, s1 = x.shape
    b_total = s0 * s1
    assert s0 % _CHUNK == 0 and s1 % 8 == 0
    nbc = s0 // _CHUNK
    nitems_total = b_total // _CHUNK
    nitems = nitems_total // _NW
    assert nitems >= 4 and nitems % 2 == 0
    x4 = (x.reshape(nbc, _CHUNK, s1 // 8, 8)
           .transpose(2, 0, 3, 1)
           .reshape(nitems_total, _CHUNK))

    mesh = plsc.VectorSubcoreMesh(core_axis_name="c", subcore_axis_name="s")
    run = functools.partial(
        pl.kernel,
        out_type=jax.ShapeDtypeStruct((s1, _D // 8, nbc, 8, _CHUNK),
                                      jnp.float32),
        mesh=mesh,
        scratch_types=[
            pltpu.VMEM((nitems, _CHUNK), jnp.int32),
            pltpu.VMEM((_CHUNK, _D), jnp.float32),
            pltpu.VMEM((_CHUNK, _D), jnp.float32),
            pltpu.VMEM((_D // 8, 8, _SKEW), jnp.float32),
            pltpu.VMEM((_D // 8, 8, _SKEW), jnp.float32),
            pltpu.SemaphoreType.DMA,
            pltpu.SemaphoreType.DMA,
            pltpu.SemaphoreType.DMA,
            pltpu.SemaphoreType.DMA,
        ],
        compiler_params=pltpu.CompilerParams(use_tc_tiling_on_sc=False,
                                             needs_layout_passes=False),
    )(functools.partial(_emb_body, nitems, nbc))
    out5 = run(x4, table)
    out = out5.transpose(2, 4, 0, 1, 3).reshape(s0, s1, _D)
    return out


# speedup vs baseline: 2.3506x; 1.3701x over previous
import functools

import jax
import jax.numpy as jnp
from jax import lax
from jax.experimental import pallas as pl
from jax.experimental.pallas import tpu as pltpu
from jax.experimental.pallas import tpu_sc as plsc

_D = 64
_SCALE = 8.0
_NC = 2
_NS = 16
_NW = _NC * _NS
_CHUNK = 128
_SKEW = 129


def _emb_body(nitems, nbc, x_hbm, tab2_hbm, out_hbm,
              idx_v, buf_a, buf_b, obuf_a, obuf_b,
              gsem_a, gsem_b, ssem_a, ssem_b):
    wid = lax.axis_index("s") * _NC + lax.axis_index("c")
    k0 = wid * nitems
    pltpu.sync_copy(x_hbm.at[pl.ds(k0, nitems)], idx_v)

    iota = lax.broadcasted_iota(jnp.int32, (16,), 0)
    zero16 = jnp.zeros((16,), jnp.int32)
    addr_cb = [(iota + 16 * cb) * _SKEW for cb in range(_D // 16)]

    def gather_copy(n, buf, gsem):
        return pltpu.make_async_copy(tab2_hbm.at[idx_v.at[n]], buf, gsem)

    def out_dst(n):
        k = k0 + n
        tt = k // (nbc * 8)
        bc = (k // 8) % nbc
        s = k % 8
        t = tt * 8 + s
        return out_hbm.at[t, :, bc]

    def scatter_copy(n, obuf, ssem):
        return pltpu.make_async_copy(
            obuf.at[:, :, pl.ds(0, _CHUNK)], out_dst(n), ssem)

    def transpose_scale(buf, obuf):
        @plsc.parallel_loop(0, _CHUNK, unroll=4)
        def rbody(r):
            l_vec = jnp.full((16,), r, jnp.int32)
            for cb in range(_D // 16):
                v = buf[r, pl.ds(cb * 16, 16)]
                plsc.store_scatter(
                    obuf, [zero16, zero16, addr_cb[cb] + l_vec], v * _SCALE)

    gather_copy(0, buf_a, gsem_a).start()
    gather_copy(0, buf_a, gsem_a).wait()
    gather_copy(1, buf_b, gsem_b).start()
    transpose_scale(buf_a, obuf_a)
    scatter_copy(0, obuf_a, ssem_a).start()

    def pair(p, _):
        n1 = 1 + 2 * p
        gather_copy(n1, buf_b, gsem_b).wait()
        gather_copy(n1 + 1, buf_a, gsem_a).start()
        scatter_copy(n1 - 1, obuf_a, ssem_a).wait()
        transpose_scale(buf_b, obuf_b)
        scatter_copy(n1, obuf_b, ssem_b).start()

        n2 = n1 + 1
        gather_copy(n2, buf_a, gsem_a).wait()
        gather_copy(n2 + 1, buf_b, gsem_b).start()
        scatter_copy(n1, obuf_b, ssem_b).wait()
        transpose_scale(buf_a, obuf_a)
        scatter_copy(n2, obuf_a, ssem_a).start()
        return 0

    lax.fori_loop(0, (nitems - 2) // 2, pair, 0)

    nl = nitems - 1
    gather_copy(nl, buf_b, gsem_b).wait()
    scatter_copy(nl - 1, obuf_a, ssem_a).wait()
    transpose_scale(buf_b, obuf_b)
    scatter_copy(nl, obuf_b, ssem_b).start()
    scatter_copy(nl, obuf_b, ssem_b).wait()


def kernel(x, table):
    s0, s1 = x.shape
    nrows = table.shape[0]
    b_total = s0 * s1
    assert s0 % _CHUNK == 0 and s1 % 8 == 0 and nrows % 2 == 0
    nbc = s0 // _CHUNK
    nitems_total = b_total // _CHUNK
    nitems = nitems_total // _NW
    assert nitems >= 4 and nitems % 2 == 0
    x4 = (x.reshape(nbc, _CHUNK, s1 // 8, 8)
           .transpose(2, 0, 3, 1)
           .reshape(nitems_total, _CHUNK))
    t2 = jnp.pad(table, ((0, 0), (0, 2 * _D - table.shape[1])))

    mesh = plsc.VectorSubcoreMesh(core_axis_name="c", subcore_axis_name="s")
    run = functools.partial(
        pl.kernel,
        out_type=jax.ShapeDtypeStruct((s1, _D // 8, nbc, 8, _CHUNK),
                                      jnp.float32),
        mesh=mesh,
        scratch_types=[
            pltpu.VMEM((nitems, _CHUNK), jnp.int32),
            pltpu.VMEM((_CHUNK, 2 * _D), jnp.float32),
            pltpu.VMEM((_CHUNK, 2 * _D), jnp.float32),
            pltpu.VMEM((_D // 8, 8, _SKEW), jnp.float32),
            pltpu.VMEM((_D // 8, 8, _SKEW), jnp.float32),
            pltpu.SemaphoreType.DMA,
            pltpu.SemaphoreType.DMA,
            pltpu.SemaphoreType.DMA,
            pltpu.SemaphoreType.DMA,
        ],
        compiler_params=pltpu.CompilerParams(use_tc_tiling_on_sc=False,
                                             needs_layout_passes=False),
    )(functools.partial(_emb_body, nitems, nbc))
    out5 = run(x4, t2)
    out = out5.transpose(2, 4, 0, 1, 3).reshape(s0, s1, _D)
    return out
